# baseline (speedup 1.0000x reference)
import functools

import numpy as np
import jax
import jax.numpy as jnp
from jax import lax
from jax.experimental import pallas as pl
from jax.experimental.pallas import tpu as pltpu
from jax.experimental.pallas import tpu_sc as plsc

B = 16384
EMBED_DIM = 32
PACK = 4
PACK_D = PACK * EMBED_DIM
NUM_BOUND = 1000
OUT_D = 7 * EMBED_DIM + 4
PAD_D = 232

NC, NS, L = 2, 16, 16
NW = NC * NS
BPW = B // NW
C = 128
NCHUNK = BPW // C
NBUF = 3

COL_U, COL_G, COL_S, COL_RE = 0, 32, 64, 96
COL_RN = 128
COL_HE, COL_HN = 161 - EMBED_DIM, 161
COL_VE, COL_VN = 194 - EMBED_DIM, 194
COL_FE, COL_FN = 227 - EMBED_DIM, 227

_NORM_DIV = float(np.sqrt(np.float32(1.0 / 12.0 + 1e-7)))

_mesh = plsc.VectorSubcoreMesh(core_axis_name="c", subcore_axis_name="s")


@functools.partial(
    pl.kernel,
    out_type=jax.ShapeDtypeStruct((B, PAD_D), jnp.float32),
    mesh=_mesh,
    compiler_params=pltpu.CompilerParams(use_tc_tiling_on_sc=False,
                                         needs_layout_passes=False),
    scratch_types=[
        pltpu.VMEM((BPW,), jnp.int32),
        pltpu.VMEM((BPW,), jnp.int32),
        pltpu.VMEM((BPW,), jnp.int32),
        pltpu.VMEM((BPW,), jnp.int32),
        pltpu.VMEM((BPW,), jnp.float32),
        pltpu.VMEM((BPW,), jnp.float32),
        pltpu.VMEM((BPW,), jnp.float32),
        pltpu.VMEM((BPW,), jnp.float32),
        pltpu.VMEM((NUM_BOUND,), jnp.float32),
        pltpu.VMEM((BPW,), jnp.int32),
        pltpu.VMEM((BPW,), jnp.int32),
        pltpu.VMEM((BPW,), jnp.int32),
        pltpu.VMEM((BPW,), jnp.int32),
        pltpu.VMEM((C, PAD_D), jnp.float32),
        pltpu.VMEM((C, PACK_D), jnp.float32),
        pltpu.VMEM((C, PACK_D), jnp.float32),
        pltpu.VMEM((C, PACK_D), jnp.float32),
        pltpu.VMEM((C, EMBED_DIM), jnp.float32),
        pltpu.VMEM((C, EMBED_DIM), jnp.float32),
        pltpu.VMEM((C, EMBED_DIM), jnp.float32),
        pltpu.VMEM((C, EMBED_DIM), jnp.float32),
        pltpu.VMEM((3, EMBED_DIM), jnp.float32),
        pltpu.VMEM((8, EMBED_DIM), jnp.float32),
        pltpu.VMEM_SHARED((NUM_BOUND + 1, EMBED_DIM), jnp.float32),
        pltpu.VMEM_SHARED((NUM_BOUND + 1, EMBED_DIM), jnp.float32),
        pltpu.VMEM_SHARED((NUM_BOUND + 1, EMBED_DIM), jnp.float32),
        pltpu.VMEM_SHARED((NUM_BOUND + 1, EMBED_DIM), jnp.float32),
        pltpu.SemaphoreType.DMA,
        pltpu.SemaphoreType.DMA,
        pltpu.SemaphoreType.DMA,
        pltpu.SemaphoreType.DMA,
    ],
)
def _sc_kernel(uid_hbm, gid_hbm, sid_hbm, f0_hbm, f1_hbm, f2_hbm, f3_hbm,
               ut_hbm, gt_hbm, st_hbm, rt_hbm, ht_hbm, vt_hbm, ft_hbm,
               bnd_hbm, out_hbm,
               up_v, uq_v, gid_v, sid_v, f0_v, f1_v, f2_v, f3_v, bnd_v,
               b0_v, b1_v, b2_v, b3_v, stage_v,
               ue0_v, ue1_v, ue2_v, re_v, he_v, ve_v, fe_v, gt_l, st_l,
               rt_sp, ht_sp, vt_sp, ft_sp, sem, sem_u0, sem_u1, sem_u2):
    sid_ax = lax.axis_index("s")
    wid = sid_ax * NC + lax.axis_index("c")
    base = wid * BPW

    @pl.when(sid_ax == 0)
    def _():
        pltpu.sync_copy(rt_hbm, rt_sp)
        pltpu.sync_copy(ht_hbm, ht_sp)
        pltpu.sync_copy(vt_hbm, vt_sp)
        pltpu.sync_copy(ft_hbm, ft_sp)

    with jax.named_scope("init_copies"):
        pltpu.sync_copy(uid_hbm.at[pl.ds(base, BPW)], up_v)
        pltpu.sync_copy(gid_hbm.at[pl.ds(base, BPW)], gid_v)
        pltpu.sync_copy(sid_hbm.at[pl.ds(base, BPW)], sid_v)
        pltpu.sync_copy(f0_hbm.at[pl.ds(base, BPW)], f0_v)
        pltpu.sync_copy(f1_hbm.at[pl.ds(base, BPW)], f1_v)
        pltpu.sync_copy(f2_hbm.at[pl.ds(base, BPW)], f2_v)
        pltpu.sync_copy(f3_hbm.at[pl.ds(base, BPW)], f3_v)
        pltpu.sync_copy(bnd_hbm, bnd_v)
        pltpu.sync_copy(gt_hbm, gt_l)
        pltpu.sync_copy(st_hbm, st_l)

    def uid_body(i, carry):
        sl = pl.ds(i * L, L)
        u = up_v[sl]
        uq_v[sl] = (u & (PACK - 1)) << 5
        up_v[sl] = lax.shift_right_logical(u, 2)
        return carry

    with jax.named_scope("uid_split"):
        lax.fori_loop(0, BPW // L, uid_body, 0)

    ue_bufs = (ue0_v, ue1_v, ue2_v)
    user_sems = (sem_u0, sem_u1, sem_u2)

    def fire_user(ch):
        return pltpu.async_copy(
            ut_hbm.at[up_v.at[pl.ds(ch * C, C)]],
            ue_bufs[ch % NBUF], user_sems[ch % NBUF])

    user_copies = [fire_user(ch) for ch in range(min(NBUF - 1, NCHUNK))]
    user_copies += [None] * (NCHUNK - len(user_copies))

    feats = ((f0_v, b0_v), (f1_v, b1_v), (f2_v, b2_v), (f3_v, b3_v))

    def bidx_body(i, carry):
        sl = pl.ds(i * L, L)
        for fref, bref in feats:
            x = fref[sl]
            c = jnp.clip((x * 999.0).astype(jnp.int32), 0, NUM_BOUND - 1)
            cm1 = jnp.maximum(c - 1, 0)
            cp1 = jnp.minimum(c + 1, NUM_BOUND - 1)
            t0 = (plsc.load_gather(bnd_v, [cm1]) < x).astype(jnp.int32)
            t1 = (plsc.load_gather(bnd_v, [c]) < x).astype(jnp.int32)
            t2 = (plsc.load_gather(bnd_v, [cp1]) < x).astype(jnp.int32)
            bref[sl] = jnp.maximum(c - 1 + t0 + t1 + t2, 0)
        return carry

    with jax.named_scope("bidx"):
        lax.fori_loop(0, BPW // L, bidx_body, 0)

    plsc.subcore_barrier()

    lanes = jnp.arange(L, dtype=jnp.int32)
    src_lo = lanes
    src_hi = lanes + L

    bsrcs = ((rt_sp, b0_v, re_v), (ht_sp, b1_v, he_v),
             (vt_sp, b2_v, ve_v), (ft_sp, b3_v, fe_v))
    fields = ((re_v, COL_RE), (he_v, COL_HE), (ve_v, COL_VE), (fe_v, COL_FE))
    dst_cols = [(col0 + lanes, col0 + lanes + L) for _, col0 in fields]
    u_cols = (COL_U + lanes, COL_U + lanes + L)
    g_cols = (COL_G + lanes, COL_G + lanes + L)
    s_cols = (COL_S + lanes, COL_S + lanes + L)

    for ch in range(NCHUNK):
        rbase = ch * C
        csl = pl.ds(rbase, C)
        bcopies = [pltpu.async_copy(tab.at[idx.at[csl]], dst, sem)
                   for tab, idx, dst in bsrcs]

        def norm_body(i, carry):
            rows = lanes + i * L
            for fref, col in ((f0_v, COL_RN), (f1_v, COL_HN),
                              (f2_v, COL_VN), (f3_v, COL_FN)):
                x = fref[pl.ds(rbase + i * L, L)]
                n = (x - 0.5) / _NORM_DIV
                plsc.store_scatter(
                    stage_v, [rows, jnp.full((L,), col, jnp.int32)], n)
            return carry

        with jax.named_scope("norm"):
            lax.fori_loop(0, C // L, norm_body, 0)

        with jax.named_scope("gather_wait"):
            for cp in bcopies:
                cp.wait()
            user_copies[ch].wait()
        if ch + NBUF - 1 < NCHUNK:
            user_copies[ch + NBUF - 1] = fire_user(ch + NBUF - 1)

        ue_v = ue_bufs[ch % NBUF]

        def asm_row(r, carry):
            rv = jnp.full((L,), r, jnp.int32)
            rgv = rv + rbase
            qv = plsc.load_gather(uq_v, [rgv])
            x = plsc.load_gather(ue_v, [rv, qv + src_lo])
            plsc.store_scatter(stage_v, [rv, u_cols[0]], x)
            y = plsc.load_gather(ue_v, [rv, qv + src_hi])
            plsc.store_scatter(stage_v, [rv, u_cols[1]], y)
            gv = plsc.load_gather(gid_v, [rgv])
            x = plsc.load_gather(gt_l, [gv, src_lo])
            plsc.store_scatter(stage_v, [rv, g_cols[0]], x)
            y = plsc.load_gather(gt_l, [gv, src_hi])
            plsc.store_scatter(stage_v, [rv, g_cols[1]], y)
            sv = plsc.load_gather(sid_v, [rgv])
            x = plsc.load_gather(st_l, [sv, src_lo])
            plsc.store_scatter(stage_v, [rv, s_cols[0]], x)
            y = plsc.load_gather(st_l, [sv, src_hi])
            plsc.store_scatter(stage_v, [rv, s_cols[1]], y)
            for (src_ref, _), (clo, chi) in zip(fields, dst_cols):
                x = plsc.load_gather(src_ref, [rv, src_lo])
                plsc.store_scatter(stage_v, [rv, clo], x)
                y = plsc.load_gather(src_ref, [rv, src_hi])
                plsc.store_scatter(stage_v, [rv, chi], y)
            return carry

        with jax.named_scope("asm"):
            lax.fori_loop(0, C, asm_row, 0)
        with jax.named_scope("out_write"):
            pltpu.sync_copy(stage_v, out_hbm.at[pl.ds(base + rbase, C), :])


def kernel(user_id, gender, status, regis_date, history, voting, favourite,
           user_table, gender_table, status_table,
           rgst_table, hsty_table, vote_table, favr_table):
    bounds = jnp.linspace(0.0, 1.0, NUM_BOUND)
    packed = user_table.reshape(user_table.shape[0] // PACK, PACK_D)
    padded = _sc_kernel(
        user_id.astype(jnp.int32), gender.astype(jnp.int32),
        status.astype(jnp.int32), regis_date, history, voting, favourite,
        packed, gender_table, status_table,
        rgst_table, hsty_table, vote_table, favr_table,
        bounds.astype(jnp.float32))
    return padded[:, :OUT_D]


# speedup vs baseline: 1.0224x; 1.0224x over previous
import functools

import numpy as np
import jax
import jax.numpy as jnp
from jax import lax
from jax.experimental import pallas as pl
from jax.experimental.pallas import tpu as pltpu
from jax.experimental.pallas import tpu_sc as plsc

B = 16384
EMBED_DIM = 32
PACK_D = 128
NUM_BOUND = 1000
OUT_D = 7 * EMBED_DIM + 4
PAD_D = 232

NC, NS, L = 2, 16, 16
NW = NC * NS
BPW = B // NW
C = 128
NCHUNK = BPW // C
NBUF = 3

COL_U, COL_G, COL_S, COL_RE = 0, 32, 64, 96
COL_RN = 128
COL_HE, COL_HN = 161 - EMBED_DIM, 161
COL_VE, COL_VN = 194 - EMBED_DIM, 194
COL_FE, COL_FN = 227 - EMBED_DIM, 227

_NORM_DIV = float(np.sqrt(np.float32(1.0 / 12.0 + 1e-7)))

_mesh = plsc.VectorSubcoreMesh(core_axis_name="c", subcore_axis_name="s")


@functools.partial(
    pl.kernel,
    out_type=jax.ShapeDtypeStruct((B, PAD_D), jnp.float32),
    mesh=_mesh,
    compiler_params=pltpu.CompilerParams(use_tc_tiling_on_sc=False,
                                         needs_layout_passes=False),
    scratch_types=[
        pltpu.VMEM((BPW,), jnp.int32),
        pltpu.VMEM((BPW,), jnp.int32),
        pltpu.VMEM((BPW,), jnp.int32),
        pltpu.VMEM((BPW,), jnp.int32),
        pltpu.VMEM((BPW,), jnp.float32),
        pltpu.VMEM((BPW,), jnp.float32),
        pltpu.VMEM((BPW,), jnp.float32),
        pltpu.VMEM((BPW,), jnp.float32),
        pltpu.VMEM((NUM_BOUND,), jnp.float32),
        pltpu.VMEM((BPW,), jnp.int32),
        pltpu.VMEM((BPW,), jnp.int32),
        pltpu.VMEM((BPW,), jnp.int32),
        pltpu.VMEM((BPW,), jnp.int32),
        pltpu.VMEM((C, PAD_D), jnp.float32),
        pltpu.VMEM((C, PACK_D), jnp.float32),
        pltpu.VMEM((C, PACK_D), jnp.float32),
        pltpu.VMEM((C, PACK_D), jnp.float32),
        pltpu.VMEM((C, EMBED_DIM), jnp.float32),
        pltpu.VMEM((C, EMBED_DIM), jnp.float32),
        pltpu.VMEM((C, EMBED_DIM), jnp.float32),
        pltpu.VMEM((C, EMBED_DIM), jnp.float32),
        pltpu.VMEM((3, EMBED_DIM), jnp.float32),
        pltpu.VMEM((8, EMBED_DIM), jnp.float32),
        pltpu.VMEM_SHARED((NUM_BOUND + 1, EMBED_DIM), jnp.float32),
        pltpu.VMEM_SHARED((NUM_BOUND + 1, EMBED_DIM), jnp.float32),
        pltpu.VMEM_SHARED((NUM_BOUND + 1, EMBED_DIM), jnp.float32),
        pltpu.VMEM_SHARED((NUM_BOUND + 1, EMBED_DIM), jnp.float32),
        pltpu.SemaphoreType.DMA,
        pltpu.SemaphoreType.DMA,
        pltpu.SemaphoreType.DMA,
        pltpu.SemaphoreType.DMA,
    ],
)
def _sc_kernel(uid_hbm, gid_hbm, sid_hbm, f0_hbm, f1_hbm, f2_hbm, f3_hbm,
               ut_hbm, gt_hbm, st_hbm, rt_hbm, ht_hbm, vt_hbm, ft_hbm,
               bnd_hbm, out_hbm,
               up_v, uq_v, gid_v, sid_v, f0_v, f1_v, f2_v, f3_v, bnd_v,
               b0_v, b1_v, b2_v, b3_v, stage_v,
               ue0_v, ue1_v, ue2_v, re_v, he_v, ve_v, fe_v, gt_l, st_l,
               rt_sp, ht_sp, vt_sp, ft_sp, sem, sem_u0, sem_u1, sem_u2):
    sid_ax = lax.axis_index("s")
    wid = sid_ax * NC + lax.axis_index("c")
    base = wid * BPW

    @pl.when(sid_ax == 0)
    def _():
        pltpu.sync_copy(rt_hbm, rt_sp)
        pltpu.sync_copy(ht_hbm, ht_sp)
        pltpu.sync_copy(vt_hbm, vt_sp)
        pltpu.sync_copy(ft_hbm, ft_sp)

    with jax.named_scope("init_copies"):
        pltpu.sync_copy(uid_hbm.at[pl.ds(base, BPW)], up_v)
        pltpu.sync_copy(gid_hbm.at[pl.ds(base, BPW)], gid_v)
        pltpu.sync_copy(sid_hbm.at[pl.ds(base, BPW)], sid_v)
        pltpu.sync_copy(f0_hbm.at[pl.ds(base, BPW)], f0_v)
        pltpu.sync_copy(f1_hbm.at[pl.ds(base, BPW)], f1_v)
        pltpu.sync_copy(f2_hbm.at[pl.ds(base, BPW)], f2_v)
        pltpu.sync_copy(f3_hbm.at[pl.ds(base, BPW)], f3_v)
        pltpu.sync_copy(bnd_hbm, bnd_v)
        pltpu.sync_copy(gt_hbm, gt_l)
        pltpu.sync_copy(st_hbm, st_l)

    ue_bufs = (ue0_v, ue1_v, ue2_v)
    user_sems = (sem_u0, sem_u1, sem_u2)

    def fire_user(ch):
        return pltpu.async_copy(
            ut_hbm.at[up_v.at[pl.ds(ch * C, C)]],
            ue_bufs[ch % NBUF], user_sems[ch % NBUF])

    user_copies = [fire_user(ch) for ch in range(min(NBUF - 1, NCHUNK))]
    user_copies += [None] * (NCHUNK - len(user_copies))

    feats = ((f0_v, b0_v), (f1_v, b1_v), (f2_v, b2_v), (f3_v, b3_v))

    def bidx_body(i, carry):
        sl = pl.ds(i * L, L)
        for fref, bref in feats:
            x = fref[sl]
            c = jnp.clip((x * 999.0).astype(jnp.int32), 0, NUM_BOUND - 1)
            cm1 = jnp.maximum(c - 1, 0)
            cp1 = jnp.minimum(c + 1, NUM_BOUND - 1)
            t0 = (plsc.load_gather(bnd_v, [cm1]) < x).astype(jnp.int32)
            t1 = (plsc.load_gather(bnd_v, [c]) < x).astype(jnp.int32)
            t2 = (plsc.load_gather(bnd_v, [cp1]) < x).astype(jnp.int32)
            bref[sl] = jnp.maximum(c - 1 + t0 + t1 + t2, 0)
        return carry

    with jax.named_scope("bidx"):
        lax.fori_loop(0, BPW // L, bidx_body, 0)

    plsc.subcore_barrier()

    lanes = jnp.arange(L, dtype=jnp.int32)
    src_lo = lanes
    src_hi = lanes + L

    bsrcs = ((rt_sp, b0_v, re_v), (ht_sp, b1_v, he_v),
             (vt_sp, b2_v, ve_v), (ft_sp, b3_v, fe_v))
    fields = ((re_v, COL_RE), (he_v, COL_HE), (ve_v, COL_VE), (fe_v, COL_FE))
    dst_cols = [(col0 + lanes, col0 + lanes + L) for _, col0 in fields]
    u_cols = (COL_U + lanes, COL_U + lanes + L)
    g_cols = (COL_G + lanes, COL_G + lanes + L)
    s_cols = (COL_S + lanes, COL_S + lanes + L)

    for ch in range(NCHUNK):
        rbase = ch * C
        csl = pl.ds(rbase, C)
        bcopies = [pltpu.async_copy(tab.at[idx.at[csl]], dst, sem)
                   for tab, idx, dst in bsrcs]

        def norm_body(i, carry):
            rows = lanes + i * L
            for fref, col in ((f0_v, COL_RN), (f1_v, COL_HN),
                              (f2_v, COL_VN), (f3_v, COL_FN)):
                x = fref[pl.ds(rbase + i * L, L)]
                n = (x - 0.5) / _NORM_DIV
                plsc.store_scatter(
                    stage_v, [rows, jnp.full((L,), col, jnp.int32)], n)
            return carry

        with jax.named_scope("norm"):
            lax.fori_loop(0, C // L, norm_body, 0)

        with jax.named_scope("gather_wait"):
            for cp in bcopies:
                cp.wait()
            user_copies[ch].wait()
        if ch + NBUF - 1 < NCHUNK:
            user_copies[ch + NBUF - 1] = fire_user(ch + NBUF - 1)

        ue_v = ue_bufs[ch % NBUF]

        def asm_row(r, carry):
            rv = jnp.full((L,), r, jnp.int32)
            rgv = rv + rbase
            x = plsc.load_gather(ue_v, [rv, src_lo])
            plsc.store_scatter(stage_v, [rv, u_cols[0]], x)
            y = plsc.load_gather(ue_v, [rv, src_hi])
            plsc.store_scatter(stage_v, [rv, u_cols[1]], y)
            gv = plsc.load_gather(gid_v, [rgv])
            x = plsc.load_gather(gt_l, [gv, src_lo])
            plsc.store_scatter(stage_v, [rv, g_cols[0]], x)
            y = plsc.load_gather(gt_l, [gv, src_hi])
            plsc.store_scatter(stage_v, [rv, g_cols[1]], y)
            sv = plsc.load_gather(sid_v, [rgv])
            x = plsc.load_gather(st_l, [sv, src_lo])
            plsc.store_scatter(stage_v, [rv, s_cols[0]], x)
            y = plsc.load_gather(st_l, [sv, src_hi])
            plsc.store_scatter(stage_v, [rv, s_cols[1]], y)
            for (src_ref, _), (clo, chi) in zip(fields, dst_cols):
                x = plsc.load_gather(src_ref, [rv, src_lo])
                plsc.store_scatter(stage_v, [rv, clo], x)
                y = plsc.load_gather(src_ref, [rv, src_hi])
                plsc.store_scatter(stage_v, [rv, chi], y)
            return carry

        with jax.named_scope("asm"):
            lax.fori_loop(0, C, asm_row, 0)
        with jax.named_scope("out_write"):
            pltpu.sync_copy(stage_v, out_hbm.at[pl.ds(base + rbase, C), :])


def kernel(user_id, gender, status, regis_date, history, voting, favourite,
           user_table, gender_table, status_table,
           rgst_table, hsty_table, vote_table, favr_table):
    bounds = jnp.linspace(0.0, 1.0, NUM_BOUND)
    packed = jnp.pad(user_table, ((0, 0), (0, PACK_D - EMBED_DIM)))
    padded = _sc_kernel(
        user_id.astype(jnp.int32), gender.astype(jnp.int32),
        status.astype(jnp.int32), regis_date, history, voting, favourite,
        packed, gender_table, status_table,
        rgst_table, hsty_table, vote_table, favr_table,
        bounds.astype(jnp.float32))
    return padded[:, :OUT_D]


# speedup vs baseline: 1.0743x; 1.0508x over previous
import functools

import numpy as np
import jax
import jax.numpy as jnp
from jax import lax
from jax.experimental import pallas as pl
from jax.experimental.pallas import tpu as pltpu
from jax.experimental.pallas import tpu_sc as plsc

B = 16384
EMBED_DIM = 32
PACK_D = 128
NUM_BOUND = 1000
OUT_D = 7 * EMBED_DIM + 4
PAD_D = 232

NC, NS, L = 2, 16, 16
NW = NC * NS
BPW = B // NW
C = 128
NCHUNK = BPW // C
NBUF = 3

COL_U, COL_G, COL_S, COL_RE = 0, 32, 64, 96
COL_RN = 128
COL_HE, COL_HN = 161 - EMBED_DIM, 161
COL_VE, COL_VN = 194 - EMBED_DIM, 194
COL_FE, COL_FN = 227 - EMBED_DIM, 227

_NORM_DIV = float(np.sqrt(np.float32(1.0 / 12.0 + 1e-7)))

_mesh = plsc.VectorSubcoreMesh(core_axis_name="c", subcore_axis_name="s")


@functools.partial(
    pl.kernel,
    out_type=jax.ShapeDtypeStruct((B, PAD_D), jnp.float32),
    mesh=_mesh,
    compiler_params=pltpu.CompilerParams(use_tc_tiling_on_sc=False,
                                         needs_layout_passes=False),
    scratch_types=[
        pltpu.VMEM((BPW,), jnp.int32),
        pltpu.VMEM((BPW,), jnp.int32),
        pltpu.VMEM((BPW,), jnp.int32),
        pltpu.VMEM((BPW,), jnp.int32),
        pltpu.VMEM((BPW,), jnp.float32),
        pltpu.VMEM((BPW,), jnp.float32),
        pltpu.VMEM((BPW,), jnp.float32),
        pltpu.VMEM((BPW,), jnp.float32),
        pltpu.VMEM((NUM_BOUND,), jnp.float32),
        pltpu.VMEM((BPW,), jnp.int32),
        pltpu.VMEM((BPW,), jnp.int32),
        pltpu.VMEM((BPW,), jnp.int32),
        pltpu.VMEM((BPW,), jnp.int32),
        pltpu.VMEM((C, PAD_D), jnp.float32),
        pltpu.VMEM((C, PACK_D), jnp.float32),
        pltpu.VMEM((C, PACK_D), jnp.float32),
        pltpu.VMEM((C, PACK_D), jnp.float32),
        pltpu.VMEM((C, EMBED_DIM), jnp.float32),
        pltpu.VMEM((C, EMBED_DIM), jnp.float32),
        pltpu.VMEM((C, EMBED_DIM), jnp.float32),
        pltpu.VMEM((C, EMBED_DIM), jnp.float32),
        pltpu.VMEM((3, EMBED_DIM), jnp.float32),
        pltpu.VMEM((8, EMBED_DIM), jnp.float32),
        pltpu.VMEM_SHARED((NUM_BOUND + 1, EMBED_DIM), jnp.float32),
        pltpu.VMEM_SHARED((NUM_BOUND + 1, EMBED_DIM), jnp.float32),
        pltpu.VMEM_SHARED((NUM_BOUND + 1, EMBED_DIM), jnp.float32),
        pltpu.VMEM_SHARED((NUM_BOUND + 1, EMBED_DIM), jnp.float32),
        pltpu.SemaphoreType.DMA,
        pltpu.SemaphoreType.DMA,
        pltpu.SemaphoreType.DMA,
        pltpu.SemaphoreType.DMA,
    ],
)
def _sc_kernel(uid_hbm, gid_hbm, sid_hbm, f0_hbm, f1_hbm, f2_hbm, f3_hbm,
               ut_hbm, gt_hbm, st_hbm, rt_hbm, ht_hbm, vt_hbm, ft_hbm,
               bnd_hbm, out_hbm,
               up_v, uq_v, gid_v, sid_v, f0_v, f1_v, f2_v, f3_v, bnd_v,
               b0_v, b1_v, b2_v, b3_v, stage_v,
               ue0_v, ue1_v, ue2_v, re_v, he_v, ve_v, fe_v, gt_l, st_l,
               rt_sp, ht_sp, vt_sp, ft_sp, sem, sem_u0, sem_u1, sem_u2):
    sid_ax = lax.axis_index("s")
    wid = sid_ax * NC + lax.axis_index("c")
    base = wid * BPW

    @pl.when(sid_ax == 0)
    def _():
        pltpu.sync_copy(rt_hbm, rt_sp)
        pltpu.sync_copy(ht_hbm, ht_sp)
        pltpu.sync_copy(vt_hbm, vt_sp)
        pltpu.sync_copy(ft_hbm, ft_sp)

    with jax.named_scope("init_copies"):
        pltpu.sync_copy(uid_hbm.at[pl.ds(base, BPW)], up_v)
        pltpu.sync_copy(gid_hbm.at[pl.ds(base, BPW)], gid_v)
        pltpu.sync_copy(sid_hbm.at[pl.ds(base, BPW)], sid_v)
        pltpu.sync_copy(f0_hbm.at[pl.ds(base, BPW)], f0_v)
        pltpu.sync_copy(f1_hbm.at[pl.ds(base, BPW)], f1_v)
        pltpu.sync_copy(f2_hbm.at[pl.ds(base, BPW)], f2_v)
        pltpu.sync_copy(f3_hbm.at[pl.ds(base, BPW)], f3_v)
        pltpu.sync_copy(bnd_hbm, bnd_v)
        pltpu.sync_copy(gt_hbm, gt_l)
        pltpu.sync_copy(st_hbm, st_l)

    ue_bufs = (ue0_v, ue1_v, ue2_v)
    user_sems = (sem_u0, sem_u1, sem_u2)

    def fire_user(ch):
        return pltpu.async_copy(
            ut_hbm.at[up_v.at[pl.ds(ch * C, C)]],
            ue_bufs[ch % NBUF], user_sems[ch % NBUF])

    user_copies = [fire_user(ch) for ch in range(min(NBUF - 1, NCHUNK))]
    user_copies += [None] * (NCHUNK - len(user_copies))

    feats = ((f0_v, b0_v), (f1_v, b1_v), (f2_v, b2_v), (f3_v, b3_v))

    def bidx_body(i, carry):
        sl = pl.ds(i * L, L)
        for fref, bref in feats:
            x = fref[sl]
            c = jnp.clip((x * 999.0).astype(jnp.int32), 0, NUM_BOUND - 1)
            cm1 = jnp.maximum(c - 1, 0)
            cp1 = jnp.minimum(c + 1, NUM_BOUND - 1)
            t0 = (plsc.load_gather(bnd_v, [cm1]) < x).astype(jnp.int32)
            t1 = (plsc.load_gather(bnd_v, [c]) < x).astype(jnp.int32)
            t2 = (plsc.load_gather(bnd_v, [cp1]) < x).astype(jnp.int32)
            bref[sl] = jnp.maximum(c - 1 + t0 + t1 + t2, 0)
        return carry

    with jax.named_scope("bidx"):
        lax.fori_loop(0, BPW // L, bidx_body, 0)

    plsc.subcore_barrier()

    lanes = jnp.arange(L, dtype=jnp.int32)
    src_lo = lanes
    src_hi = lanes + L

    bsrcs = ((rt_sp, b0_v, re_v), (ht_sp, b1_v, he_v),
             (vt_sp, b2_v, ve_v), (ft_sp, b3_v, fe_v))
    fields = ((re_v, COL_RE), (he_v, COL_HE), (ve_v, COL_VE), (fe_v, COL_FE))
    dst_cols = [(col0 + lanes, col0 + lanes + L) for _, col0 in fields]
    u_cols = (COL_U + lanes, COL_U + lanes + L)
    g_cols = (COL_G + lanes, COL_G + lanes + L)
    s_cols = (COL_S + lanes, COL_S + lanes + L)

    for ch in range(NCHUNK):
        rbase = ch * C
        csl = pl.ds(rbase, C)
        bcopies = [pltpu.async_copy(tab.at[idx.at[csl]], dst, sem)
                   for tab, idx, dst in bsrcs]

        def norm_body(i, carry):
            rows = lanes + i * L
            for fref, col in ((f0_v, COL_RN), (f1_v, COL_HN),
                              (f2_v, COL_VN), (f3_v, COL_FN)):
                x = fref[pl.ds(rbase + i * L, L)]
                n = (x - 0.5) / _NORM_DIV
                plsc.store_scatter(
                    stage_v, [rows, jnp.full((L,), col, jnp.int32)], n)
            return carry

        with jax.named_scope("norm"):
            plsc.parallel_loop(0, C // L, 1, unroll=2)(
                lambda i: norm_body(i, 0))

        with jax.named_scope("gather_wait"):
            for cp in bcopies:
                cp.wait()
            user_copies[ch].wait()
        if ch + NBUF - 1 < NCHUNK:
            user_copies[ch + NBUF - 1] = fire_user(ch + NBUF - 1)

        ue_v = ue_bufs[ch % NBUF]

        def asm_row(r, carry):
            rv = jnp.full((L,), r, jnp.int32)
            rgv = rv + rbase
            x = plsc.load_gather(ue_v, [rv, src_lo])
            plsc.store_scatter(stage_v, [rv, u_cols[0]], x)
            y = plsc.load_gather(ue_v, [rv, src_hi])
            plsc.store_scatter(stage_v, [rv, u_cols[1]], y)
            gv = plsc.load_gather(gid_v, [rgv])
            x = plsc.load_gather(gt_l, [gv, src_lo])
            plsc.store_scatter(stage_v, [rv, g_cols[0]], x)
            y = plsc.load_gather(gt_l, [gv, src_hi])
            plsc.store_scatter(stage_v, [rv, g_cols[1]], y)
            sv = plsc.load_gather(sid_v, [rgv])
            x = plsc.load_gather(st_l, [sv, src_lo])
            plsc.store_scatter(stage_v, [rv, s_cols[0]], x)
            y = plsc.load_gather(st_l, [sv, src_hi])
            plsc.store_scatter(stage_v, [rv, s_cols[1]], y)
            for (src_ref, _), (clo, chi) in zip(fields, dst_cols):
                x = plsc.load_gather(src_ref, [rv, src_lo])
                plsc.store_scatter(stage_v, [rv, clo], x)
                y = plsc.load_gather(src_ref, [rv, src_hi])
                plsc.store_scatter(stage_v, [rv, chi], y)
            return carry

        with jax.named_scope("asm"):
            plsc.parallel_loop(0, C, 1, unroll=4)(
                lambda r: asm_row(r, 0))
        with jax.named_scope("out_write"):
            pltpu.sync_copy(stage_v, out_hbm.at[pl.ds(base + rbase, C), :])


def kernel(user_id, gender, status, regis_date, history, voting, favourite,
           user_table, gender_table, status_table,
           rgst_table, hsty_table, vote_table, favr_table):
    bounds = jnp.linspace(0.0, 1.0, NUM_BOUND)
    packed = jnp.pad(user_table, ((0, 0), (0, PACK_D - EMBED_DIM)))
    padded = _sc_kernel(
        user_id.astype(jnp.int32), gender.astype(jnp.int32),
        status.astype(jnp.int32), regis_date, history, voting, favourite,
        packed, gender_table, status_table,
        rgst_table, hsty_table, vote_table, favr_table,
        bounds.astype(jnp.float32))
    return padded[:, :OUT_D]


# speedup vs baseline: 1.0816x; 1.0068x over previous
import functools

import numpy as np
import jax
import jax.numpy as jnp
from jax import lax
from jax.experimental import pallas as pl
from jax.experimental.pallas import tpu as pltpu
from jax.experimental.pallas import tpu_sc as plsc

B = 16384
EMBED_DIM = 32
PACK_D = 128
NUM_BOUND = 1000
OUT_D = 7 * EMBED_DIM + 4
PAD_D = 232

NC, NS, L = 2, 16, 16
NW = NC * NS
BPW = B // NW
C = 128
NCHUNK = BPW // C
NBUF = 3

COL_U, COL_G, COL_S, COL_RE = 0, 32, 64, 96
COL_RN = 128
COL_HE, COL_HN = 161 - EMBED_DIM, 161
COL_VE, COL_VN = 194 - EMBED_DIM, 194
COL_FE, COL_FN = 227 - EMBED_DIM, 227

_NORM_DIV = float(np.sqrt(np.float32(1.0 / 12.0 + 1e-7)))

_mesh = plsc.VectorSubcoreMesh(core_axis_name="c", subcore_axis_name="s")


@functools.partial(
    pl.kernel,
    out_type=jax.ShapeDtypeStruct((B, PAD_D), jnp.float32),
    mesh=_mesh,
    compiler_params=pltpu.CompilerParams(use_tc_tiling_on_sc=False,
                                         needs_layout_passes=False),
    scratch_types=[
        pltpu.VMEM((BPW,), jnp.int32),
        pltpu.VMEM((BPW,), jnp.int32),
        pltpu.VMEM((BPW,), jnp.int32),
        pltpu.VMEM((BPW,), jnp.float32),
        pltpu.VMEM((BPW,), jnp.float32),
        pltpu.VMEM((BPW,), jnp.float32),
        pltpu.VMEM((BPW,), jnp.float32),
        pltpu.VMEM((NUM_BOUND,), jnp.float32),
        pltpu.VMEM((BPW,), jnp.int32),
        pltpu.VMEM((BPW,), jnp.int32),
        pltpu.VMEM((BPW,), jnp.int32),
        pltpu.VMEM((BPW,), jnp.int32),
        pltpu.VMEM((C, PAD_D), jnp.float32),
        pltpu.VMEM((C, PACK_D), jnp.float32),
        pltpu.VMEM((C, PACK_D), jnp.float32),
        pltpu.VMEM((C, PACK_D), jnp.float32),
        pltpu.VMEM((C, EMBED_DIM), jnp.float32),
        pltpu.VMEM((C, EMBED_DIM), jnp.float32),
        pltpu.VMEM((C, EMBED_DIM), jnp.float32),
        pltpu.VMEM((C, EMBED_DIM), jnp.float32),
        pltpu.VMEM((3, EMBED_DIM), jnp.float32),
        pltpu.VMEM((8, EMBED_DIM), jnp.float32),
        pltpu.VMEM_SHARED((NUM_BOUND + 1, EMBED_DIM), jnp.float32),
        pltpu.VMEM_SHARED((NUM_BOUND + 1, EMBED_DIM), jnp.float32),
        pltpu.VMEM_SHARED((NUM_BOUND + 1, EMBED_DIM), jnp.float32),
        pltpu.VMEM_SHARED((NUM_BOUND + 1, EMBED_DIM), jnp.float32),
        pltpu.SemaphoreType.DMA,
        pltpu.SemaphoreType.DMA,
        pltpu.SemaphoreType.DMA,
        pltpu.SemaphoreType.DMA,
    ],
)
def _sc_kernel(uid_hbm, gid_hbm, sid_hbm, f0_hbm, f1_hbm, f2_hbm, f3_hbm,
               ut_hbm, gt_hbm, st_hbm, rt_hbm, ht_hbm, vt_hbm, ft_hbm,
               bnd_hbm, out_hbm,
               up_v, gid_v, sid_v, f0_v, f1_v, f2_v, f3_v, bnd_v,
               b0_v, b1_v, b2_v, b3_v, stage_v,
               ue0_v, ue1_v, ue2_v, re_v, he_v, ve_v, fe_v, gt_l, st_l,
               rt_sp, ht_sp, vt_sp, ft_sp, sem, sem_u0, sem_u1, sem_u2):
    sid_ax = lax.axis_index("s")
    wid = sid_ax * NC + lax.axis_index("c")
    base = wid * BPW

    @pl.when(sid_ax == 0)
    def _():
        pltpu.sync_copy(rt_hbm, rt_sp)
        pltpu.sync_copy(ht_hbm, ht_sp)
        pltpu.sync_copy(vt_hbm, vt_sp)
        pltpu.sync_copy(ft_hbm, ft_sp)

    with jax.named_scope("init_copies"):
        init = [
            pltpu.async_copy(uid_hbm.at[pl.ds(base, BPW)], up_v, sem),
            pltpu.async_copy(gid_hbm.at[pl.ds(base, BPW)], gid_v, sem),
            pltpu.async_copy(sid_hbm.at[pl.ds(base, BPW)], sid_v, sem),
            pltpu.async_copy(f0_hbm.at[pl.ds(base, BPW)], f0_v, sem),
            pltpu.async_copy(f1_hbm.at[pl.ds(base, BPW)], f1_v, sem),
            pltpu.async_copy(f2_hbm.at[pl.ds(base, BPW)], f2_v, sem),
            pltpu.async_copy(f3_hbm.at[pl.ds(base, BPW)], f3_v, sem),
            pltpu.async_copy(bnd_hbm, bnd_v, sem),
            pltpu.async_copy(gt_hbm, gt_l, sem),
            pltpu.async_copy(st_hbm, st_l, sem),
        ]
        for cp in init:
            cp.wait()

    ue_bufs = (ue0_v, ue1_v, ue2_v)
    user_sems = (sem_u0, sem_u1, sem_u2)

    def fire_user(ch):
        return pltpu.async_copy(
            ut_hbm.at[up_v.at[pl.ds(ch * C, C)]],
            ue_bufs[ch % NBUF], user_sems[ch % NBUF])

    user_copies = [fire_user(ch) for ch in range(min(NBUF - 1, NCHUNK))]
    user_copies += [None] * (NCHUNK - len(user_copies))

    feats = ((f0_v, b0_v), (f1_v, b1_v), (f2_v, b2_v), (f3_v, b3_v))

    def bidx_body(i, carry):
        sl = pl.ds(i * L, L)
        for fref, bref in feats:
            x = fref[sl]
            c = jnp.clip((x * 999.0).astype(jnp.int32), 0, NUM_BOUND - 1)
            cm1 = jnp.maximum(c - 1, 0)
            cp1 = jnp.minimum(c + 1, NUM_BOUND - 1)
            t0 = (plsc.load_gather(bnd_v, [cm1]) < x).astype(jnp.int32)
            t1 = (plsc.load_gather(bnd_v, [c]) < x).astype(jnp.int32)
            t2 = (plsc.load_gather(bnd_v, [cp1]) < x).astype(jnp.int32)
            bref[sl] = jnp.maximum(c - 1 + t0 + t1 + t2, 0)
        return carry

    with jax.named_scope("bidx"):
        lax.fori_loop(0, BPW // L, bidx_body, 0)

    plsc.subcore_barrier()

    lanes = jnp.arange(L, dtype=jnp.int32)
    src_lo = lanes
    src_hi = lanes + L

    bsrcs = ((rt_sp, b0_v, re_v), (ht_sp, b1_v, he_v),
             (vt_sp, b2_v, ve_v), (ft_sp, b3_v, fe_v))
    fields = ((re_v, COL_RE), (he_v, COL_HE), (ve_v, COL_VE), (fe_v, COL_FE))
    dst_cols = [(col0 + lanes, col0 + lanes + L) for _, col0 in fields]
    u_cols = (COL_U + lanes, COL_U + lanes + L)
    g_cols = (COL_G + lanes, COL_G + lanes + L)
    s_cols = (COL_S + lanes, COL_S + lanes + L)

    for ch in range(NCHUNK):
        rbase = ch * C
        csl = pl.ds(rbase, C)
        bcopies = [pltpu.async_copy(tab.at[idx.at[csl]], dst, sem)
                   for tab, idx, dst in bsrcs]

        def norm_body(i, carry):
            rows = lanes + i * L
            for fref, col in ((f0_v, COL_RN), (f1_v, COL_HN),
                              (f2_v, COL_VN), (f3_v, COL_FN)):
                x = fref[pl.ds(rbase + i * L, L)]
                n = (x - 0.5) / _NORM_DIV
                plsc.store_scatter(
                    stage_v, [rows, jnp.full((L,), col, jnp.int32)], n)
            return carry

        with jax.named_scope("norm"):
            plsc.parallel_loop(0, C // L, 1, unroll=2)(
                lambda i: norm_body(i, 0))

        with jax.named_scope("gather_wait"):
            for cp in bcopies:
                cp.wait()
            user_copies[ch].wait()
        if ch + NBUF - 1 < NCHUNK:
            user_copies[ch + NBUF - 1] = fire_user(ch + NBUF - 1)

        ue_v = ue_bufs[ch % NBUF]

        def asm_row(r, carry):
            rv = jnp.full((L,), r, jnp.int32)
            rgv = rv + rbase
            x = plsc.load_gather(ue_v, [rv, src_lo])
            plsc.store_scatter(stage_v, [rv, u_cols[0]], x)
            y = plsc.load_gather(ue_v, [rv, src_hi])
            plsc.store_scatter(stage_v, [rv, u_cols[1]], y)
            gv = plsc.load_gather(gid_v, [rgv])
            x = plsc.load_gather(gt_l, [gv, src_lo])
            plsc.store_scatter(stage_v, [rv, g_cols[0]], x)
            y = plsc.load_gather(gt_l, [gv, src_hi])
            plsc.store_scatter(stage_v, [rv, g_cols[1]], y)
            sv = plsc.load_gather(sid_v, [rgv])
            x = plsc.load_gather(st_l, [sv, src_lo])
            plsc.store_scatter(stage_v, [rv, s_cols[0]], x)
            y = plsc.load_gather(st_l, [sv, src_hi])
            plsc.store_scatter(stage_v, [rv, s_cols[1]], y)
            for (src_ref, _), (clo, chi) in zip(fields, dst_cols):
                x = plsc.load_gather(src_ref, [rv, src_lo])
                plsc.store_scatter(stage_v, [rv, clo], x)
                y = plsc.load_gather(src_ref, [rv, src_hi])
                plsc.store_scatter(stage_v, [rv, chi], y)
            return carry

        with jax.named_scope("asm"):
            plsc.parallel_loop(0, C, 1, unroll=8)(
                lambda r: asm_row(r, 0))
        with jax.named_scope("out_write"):
            pltpu.sync_copy(stage_v, out_hbm.at[pl.ds(base + rbase, C), :])


def kernel(user_id, gender, status, regis_date, history, voting, favourite,
           user_table, gender_table, status_table,
           rgst_table, hsty_table, vote_table, favr_table):
    bounds = jnp.linspace(0.0, 1.0, NUM_BOUND)
    packed = jnp.pad(user_table, ((0, 0), (0, PACK_D - EMBED_DIM)))
    padded = _sc_kernel(
        user_id.astype(jnp.int32), gender.astype(jnp.int32),
        status.astype(jnp.int32), regis_date, history, voting, favourite,
        packed, gender_table, status_table,
        rgst_table, hsty_table, vote_table, favr_table,
        bounds.astype(jnp.float32))
    return padded[:, :OUT_D]
